# Initial kernel scaffold; baseline (speedup 1.0000x reference)
import jax
import jax.numpy as jnp
from jax.experimental import pallas as pl


def kernel(x, node_rep, edge_index, attention, W1, b1, W2, b2):
    raise NotImplementedError("write your pallas kernel here")


# speedup vs baseline: 4.3849x; 4.3849x over previous
import functools

import jax
import jax.numpy as jnp
import numpy as np
from jax.experimental import pallas as pl

N = 10000
E = 320000
DIM = 128
NB = 4
BD = 4
GRP = DIM // (NB * BD)
HID = 256
TIME = 21
NUM_ATT = 5
NP_ = 10240
BM = 256
BMM = 256


def _build_perm_mats():
    Ps = np.zeros((BD, DIM, DIM), dtype=np.float32)
    for d in range(BD):
        for b in range(NB):
            for c in range(BD):
                for e in range(GRP):
                    src_col = b * BD * GRP + d * GRP + e
                    dst_col = b * BD * GRP + c * GRP + e
                    Ps[d, src_col, dst_col] = 1.0
    Bm = np.zeros((NB * BD, DIM), dtype=np.float32)
    for b in range(NB):
        for c in range(BD):
            for e in range(GRP):
                Bm[b * BD + c, b * BD * GRP + c * GRP + e] = 1.0
    return jnp.asarray(Ps.reshape(BD * DIM, DIM)), jnp.asarray(Bm)


def _block_transform(h, s_cat, ps_ref, b_ref):
    out = jnp.zeros_like(h)
    bmat = b_ref[...]
    for d in range(BD):
        p_d = ps_ref[d * DIM:(d + 1) * DIM, :]
        s_d = s_cat[:, d * NB * BD:(d + 1) * NB * BD]
        out = out + jnp.dot(h, p_d, preferred_element_type=jnp.float32) * \
            jnp.dot(s_d, bmat, preferred_element_type=jnp.float32)
    return out


def _head_kernel(x_ref, s_ref, ps_ref, b_ref, o_ref):
    o_ref[...] = _block_transform(x_ref[...], s_ref[...], ps_ref, b_ref)


def _matmul_kernel(a_ref, h_ref, o_ref):
    o_ref[...] = jnp.dot(a_ref[...], h_ref[...],
                         preferred_element_type=jnp.float32)


def _tail_kernel(h1_ref, h2_ref, h5_ref, h20_ref, att_ref, w1_ref, b1_ref,
                 w2_ref, b2_ref, m_ref, ps_ref, b_ref, o_ref):
    att = att_ref[...]
    h = (h1_ref[...] * att[0:1, :] + h2_ref[...] * att[1:2, :] +
         h5_ref[...] * att[2:3, :] + h20_ref[...] * att[3:4, :])
    hid = jnp.dot(h, w1_ref[...], preferred_element_type=jnp.float32) \
        + b1_ref[...]
    hid = 0.5 * hid * (1.0 + jax.lax.erf(hid * np.float32(0.7071067811865476)))
    h = jnp.dot(hid, w2_ref[...], preferred_element_type=jnp.float32) \
        + b2_ref[...]
    o_ref[...] = _block_transform(h, m_ref[...], ps_ref, b_ref)


def _propagate(a_pad, h):
    grid = (NP_ // BMM,)
    return pl.pallas_call(
        _matmul_kernel,
        grid=grid,
        in_specs=[
            pl.BlockSpec((BMM, NP_), lambda i: (i, 0)),
            pl.BlockSpec((NP_, DIM), lambda i: (0, 0)),
        ],
        out_specs=pl.BlockSpec((BMM, DIM), lambda i: (i, 0)),
        out_shape=jax.ShapeDtypeStruct((NP_, DIM), jnp.float32),
    )(a_pad, h)


@jax.jit
def _run(x, node_rep, edge_index, attention, W1, b1, W2, b2):
    src = edge_index[0].astype(jnp.int32)
    dst = edge_index[1].astype(jnp.int32)

    degrees = jnp.zeros((N,), jnp.float32).at[src].add(1.0)
    norm_coefs = 1.0 / jnp.sqrt(degrees[src] * degrees[dst])
    a_pad = jnp.zeros((NP_, NP_), jnp.float32).at[dst, src].add(norm_coefs)

    ps_mat, b_mat = _build_perm_mats()

    x_pad = jnp.zeros((NP_, DIM), jnp.float32).at[:N].set(x)
    s_cat = jnp.concatenate(
        [node_rep[:, :, :, d].reshape(N, NB * BD) for d in range(BD)], axis=1)
    s_cat = jnp.zeros((NP_, BD * NB * BD), jnp.float32).at[:N].set(s_cat)
    m_cat = jnp.concatenate(
        [node_rep[:, :, d, :].reshape(N, NB * BD) for d in range(BD)], axis=1)
    m_cat = jnp.zeros((NP_, BD * NB * BD), jnp.float32).at[:N].set(m_cat)

    grid = (NP_ // BM,)
    h = pl.pallas_call(
        _head_kernel,
        grid=grid,
        in_specs=[
            pl.BlockSpec((BM, DIM), lambda i: (i, 0)),
            pl.BlockSpec((BM, BD * NB * BD), lambda i: (i, 0)),
            pl.BlockSpec((BD * DIM, DIM), lambda i: (0, 0)),
            pl.BlockSpec((NB * BD, DIM), lambda i: (0, 0)),
        ],
        out_specs=pl.BlockSpec((BM, DIM), lambda i: (i, 0)),
        out_shape=jax.ShapeDtypeStruct((NP_, DIM), jnp.float32),
    )(x_pad, s_cat, ps_mat, b_mat)

    captured = {}
    for t in range(1, TIME + 1):
        h = _propagate(a_pad, h)
        if t in (1, 2, 5, 20):
            captured[t] = h

    att = jax.nn.softmax(attention, axis=0)
    att2d = jnp.broadcast_to(att[:4, None], (4, DIM)).astype(jnp.float32)

    message = pl.pallas_call(
        _tail_kernel,
        grid=grid,
        in_specs=[
            pl.BlockSpec((BM, DIM), lambda i: (i, 0)),
            pl.BlockSpec((BM, DIM), lambda i: (i, 0)),
            pl.BlockSpec((BM, DIM), lambda i: (i, 0)),
            pl.BlockSpec((BM, DIM), lambda i: (i, 0)),
            pl.BlockSpec((4, DIM), lambda i: (0, 0)),
            pl.BlockSpec((DIM, HID), lambda i: (0, 0)),
            pl.BlockSpec((1, HID), lambda i: (0, 0)),
            pl.BlockSpec((HID, DIM), lambda i: (0, 0)),
            pl.BlockSpec((1, DIM), lambda i: (0, 0)),
            pl.BlockSpec((BM, BD * NB * BD), lambda i: (i, 0)),
            pl.BlockSpec((BD * DIM, DIM), lambda i: (0, 0)),
            pl.BlockSpec((NB * BD, DIM), lambda i: (0, 0)),
        ],
        out_specs=pl.BlockSpec((BM, DIM), lambda i: (i, 0)),
        out_shape=jax.ShapeDtypeStruct((NP_, DIM), jnp.float32),
    )(captured[1], captured[2], captured[5], captured[20], att2d,
      W1, b1.reshape(1, HID), W2, b2.reshape(1, DIM), m_cat, ps_mat, b_mat)

    return jnp.concatenate([x, message[:N]], axis=1)


def kernel(x, node_rep, edge_index, attention, W1, b1, W2, b2):
    return _run(x, node_rep, edge_index, attention, W1, b1, W2, b2)
